# baseline (speedup 1.0000x reference)
import functools
import math

import jax
import jax.numpy as jnp
from jax import lax
from jax.experimental import pallas as pl
from jax.experimental.pallas import tpu as pltpu
from jax.experimental.pallas import tpu_sc as plsc

DIM = 1024
E = 8
K = 2
HID = 4096
T = 2048
TT = 128
NTT = T // TT
TTG = 256
HB = 1024
NHB = HID // HB
NP = K * T
NT = 24
NR = NT * TTG
EPS = 1e-6
_INV_SQRT2 = 1.0 / math.sqrt(2.0)

_NC = 2
_NW = 32
_TOKW = T // _NW
_CH = 16


def _router_body(x_ref, rg_ref, rb_ref, rw_ref, rwb_ref,
                 xhat_ref, idx_ref, wn_ref):
    x = x_ref[...]
    mu = jnp.mean(x, axis=1, keepdims=True)
    xc = x - mu
    var = jnp.mean(xc * xc, axis=1, keepdims=True)
    xh = xc * lax.rsqrt(var + EPS)
    xhat_ref[...] = xh
    xr = xh * rg_ref[...] + rb_ref[...]
    l8 = jax.lax.dot_general(rw_ref[...], xr, (((1,), (1,)), ((), ())),
                             preferred_element_type=jnp.float32)
    l8 = l8 + rwb_ref[...]
    m = jnp.max(l8, axis=0, keepdims=True)
    pe = jnp.exp(l8 - m)
    p = pe / jnp.sum(pe, axis=0, keepdims=True)
    io = lax.broadcasted_iota(jnp.int32, (E, TT), 0)
    m1 = jnp.max(p, axis=0, keepdims=True)
    i1 = jnp.min(jnp.where(p >= m1, io, E), axis=0, keepdims=True)
    p2 = jnp.where(io == i1, -jnp.inf, p)
    m2 = jnp.max(p2, axis=0, keepdims=True)
    i2 = jnp.min(jnp.where(p2 >= m2, io, E), axis=0, keepdims=True)
    s = m1 + m2
    idx_ref[...] = jnp.concatenate([i1, i2], axis=0)
    wn_ref[...] = jnp.concatenate([m1 / s, m2 / s], axis=0)


def _router_call(xf, r_g, r_b, r_w, r_wb):
    return pl.pallas_call(
        _router_body,
        grid=(NTT,),
        in_specs=[
            pl.BlockSpec((TT, DIM), lambda t: (t, 0)),
            pl.BlockSpec((1, DIM), lambda t: (0, 0)),
            pl.BlockSpec((1, DIM), lambda t: (0, 0)),
            pl.BlockSpec((E, DIM), lambda t: (0, 0)),
            pl.BlockSpec((E, 1), lambda t: (0, 0)),
        ],
        out_specs=[
            pl.BlockSpec((TT, DIM), lambda t: (t, 0)),
            pl.BlockSpec((K, TT), lambda t: (0, t)),
            pl.BlockSpec((K, TT), lambda t: (0, t)),
        ],
        out_shape=[
            jax.ShapeDtypeStruct((T, DIM), jnp.float32),
            jax.ShapeDtypeStruct((K, T), jnp.int32),
            jax.ShapeDtypeStruct((K, T), jnp.float32),
        ],
        compiler_params=pltpu.CompilerParams(
            dimension_semantics=("arbitrary",)),
    )(xf, r_g.reshape(1, DIM), r_b.reshape(1, DIM), r_w, r_wb.reshape(E, 1))


_PR = 32
_PC = NP // _PR


def _plan_body(ep_ref, dst_ref, meta_ref):
    ep = ep_ref[...]
    hp = jax.lax.Precision.HIGHEST
    ltri = (lax.broadcasted_iota(jnp.int32, (_PC, _PC), 0)
            <= lax.broadcasted_iota(jnp.int32, (_PC, _PC), 1)
            ).astype(jnp.float32)
    strt = (lax.broadcasted_iota(jnp.int32, (_PR, _PR), 1)
            < lax.broadcasted_iota(jnp.int32, (_PR, _PR), 0)
            ).astype(jnp.float32)
    dst = jnp.zeros((_PR, _PC), jnp.float32)
    cnts = []
    for e in range(E):
        m_e = (ep == e).astype(jnp.float32)
        local = jax.lax.dot_general(m_e, ltri, (((1,), (0,)), ((), ())),
                                    precision=hp,
                                    preferred_element_type=jnp.float32)
        rowsum = jnp.sum(m_e, axis=1, keepdims=True)
        carry = jax.lax.dot_general(strt, rowsum, (((1,), (0,)), ((), ())),
                                    precision=hp,
                                    preferred_element_type=jnp.float32)
        cnts.append(jnp.sum(rowsum))
        dst = dst + m_e * (local - 1.0 + carry)
    off = []
    run = jnp.float32(0.0)
    total = jnp.float32(0.0)
    elast = jnp.float32(0.0)
    for e in range(E):
        off.append(run)
        pc = jnp.ceil(cnts[e] * (1.0 / TTG)) * TTG
        run = run + pc
        total = total + pc
        elast = jnp.where(cnts[e] > 0, jnp.float32(e), elast)
    for e in range(E):
        m_e = (ep == e).astype(jnp.float32)
        dst = dst + m_e * off[e]
    dst_ref[...] = dst.astype(jnp.int32)
    starts = lax.broadcasted_iota(jnp.int32, (1, TT), 1) * TTG
    te = jnp.zeros((1, TT), jnp.int32)
    for e in range(E):
        te = te + (starts >= off[e].astype(jnp.int32)).astype(jnp.int32)
    te = jnp.minimum(te - 1, elast.astype(jnp.int32))
    tv = (starts < total.astype(jnp.int32)).astype(jnp.int32)
    meta_ref[...] = jnp.concatenate([te, tv], axis=0)


def _plan_call(e_pairs):
    return pl.pallas_call(
        _plan_body,
        grid=(1,),
        in_specs=[pl.BlockSpec((_PR, _PC), lambda i: (0, 0))],
        out_specs=[
            pl.BlockSpec((_PR, _PC), lambda i: (0, 0)),
            pl.BlockSpec((2, TT), lambda i: (0, 0)),
        ],
        out_shape=[
            jax.ShapeDtypeStruct((_PR, _PC), jnp.int32),
            jax.ShapeDtypeStruct((2, TT), jnp.int32),
        ],
        compiler_params=pltpu.CompilerParams(
            dimension_semantics=("arbitrary",)),
    )(e_pairs)


def _sc_scatter(xhat, dst):
    mesh = plsc.VectorSubcoreMesh(core_axis_name="c", subcore_axis_name="s")

    @functools.partial(
        pl.kernel, mesh=mesh,
        out_type=jax.ShapeDtypeStruct((NR, DIM), jnp.float32),
        scratch_types=[
            pltpu.VMEM((_CH, DIM), jnp.float32),
            pltpu.VMEM((_CH,), jnp.int32),
            pltpu.VMEM((_CH,), jnp.int32),
            pltpu.SemaphoreType.DMA,
        ],
    )
    def k(xhat_hbm, dst_hbm, xs_hbm, rows_v, i0_v, i1_v, sem):
        wid = lax.axis_index("s") * _NC + lax.axis_index("c")
        base = wid * _TOKW
        for c in range(_TOKW // _CH):
            tb = base + c * _CH
            pltpu.sync_copy(xhat_hbm.at[pl.ds(tb, _CH)], rows_v)
            pltpu.sync_copy(dst_hbm.at[pl.ds(tb, _CH)], i0_v)
            pltpu.sync_copy(dst_hbm.at[pl.ds(T + tb, _CH)], i1_v)
            pltpu.async_copy(rows_v, xs_hbm.at[i0_v], sem).wait()
            pltpu.async_copy(rows_v, xs_hbm.at[i1_v], sem).wait()

    return k(xhat, dst)


def _grouped_body(s_ref, xs_ref, w1_ref, b1_ref, w2_ref, b2_ref,
                  lng_ref, lnb_ref, ys_ref):
    h = pl.program_id(0)
    t = pl.program_id(1)
    rows = pl.ds(t * TTG, TTG)

    @pl.when(s_ref[NT + t] != 0)
    def _compute():
        xt = xs_ref[...]
        xln = (xt * lng_ref[0] + lnb_ref[0]).astype(jnp.bfloat16)
        hm = jax.lax.dot_general(xln, w1_ref[0],
                                 (((1,), (1,)), ((), ())),
                                 preferred_element_type=jnp.float32)
        hm = hm + b1_ref[0]
        hg = 0.5 * hm * (1.0 + lax.erf(hm * _INV_SQRT2))
        contrib = jax.lax.dot_general(hg.astype(jnp.bfloat16),
                                      w2_ref[0],
                                      (((1,), (1,)), ((), ())),
                                      preferred_element_type=jnp.float32)

        @pl.when(h == 0)
        def _init():
            ys_ref[rows, :] = contrib + b2_ref[0]

        @pl.when(h != 0)
        def _acc():
            ys_ref[rows, :] = ys_ref[rows, :] + contrib


def _grouped_call(sarr, xs, ln_g, ln_b, W1, b1, W2, b2):
    grid_spec = pltpu.PrefetchScalarGridSpec(
        num_scalar_prefetch=1,
        grid=(NHB, NT),
        in_specs=[
            pl.BlockSpec((TTG, DIM), lambda h, t, s: (t, 0)),
            pl.BlockSpec((1, HB, DIM), lambda h, t, s: (s[t], h, 0)),
            pl.BlockSpec((1, 1, HB), lambda h, t, s: (s[t], 0, h)),
            pl.BlockSpec((1, DIM, HB), lambda h, t, s: (s[t], 0, h)),
            pl.BlockSpec((1, 1, DIM), lambda h, t, s: (s[t], 0, 0)),
            pl.BlockSpec((1, 1, DIM), lambda h, t, s: (s[t], 0, 0)),
            pl.BlockSpec((1, 1, DIM), lambda h, t, s: (s[t], 0, 0)),
        ],
        out_specs=pl.BlockSpec((NR, DIM), lambda h, t, s: (0, 0)),
    )
    return pl.pallas_call(
        _grouped_body,
        grid_spec=grid_spec,
        out_shape=jax.ShapeDtypeStruct((NR, DIM), jnp.float32),
        compiler_params=pltpu.CompilerParams(
            dimension_semantics=("arbitrary", "arbitrary")),
    )(sarr, xs, W1, b1.reshape(E, 1, HID), W2, b2.reshape(E, 1, DIM),
      ln_g.reshape(E, 1, DIM), ln_b.reshape(E, 1, DIM))


def _sc_combine(ys, dst, g16):
    mesh = plsc.VectorSubcoreMesh(core_axis_name="c", subcore_axis_name="s")

    @functools.partial(
        pl.kernel, mesh=mesh,
        out_type=jax.ShapeDtypeStruct((T, DIM), jnp.float32),
        scratch_types=[
            pltpu.VMEM((_CH, DIM), jnp.float32),
            pltpu.VMEM((_CH, DIM), jnp.float32),
            pltpu.VMEM((_CH, DIM), jnp.float32),
            pltpu.VMEM((_CH,), jnp.int32),
            pltpu.VMEM((_CH,), jnp.int32),
            pltpu.VMEM((_CH, 16), jnp.float32),
            pltpu.VMEM((_CH, 16), jnp.float32),
            pltpu.SemaphoreType.DMA,
        ],
    )
    def k(ys_hbm, dst_hbm, g_hbm, out_hbm,
          ra_v, rb_v, o_v, i0_v, i1_v, g0_v, g1_v, sem):
        wid = lax.axis_index("s") * _NC + lax.axis_index("c")
        base = wid * _TOKW
        for c in range(_TOKW // _CH):
            tb = base + c * _CH
            pltpu.sync_copy(dst_hbm.at[pl.ds(tb, _CH)], i0_v)
            pltpu.sync_copy(dst_hbm.at[pl.ds(T + tb, _CH)], i1_v)
            pltpu.sync_copy(g_hbm.at[pl.ds(tb, _CH)], g0_v)
            pltpu.sync_copy(g_hbm.at[pl.ds(T + tb, _CH)], g1_v)
            pltpu.async_copy(ys_hbm.at[i0_v], ra_v, sem).wait()
            pltpu.async_copy(ys_hbm.at[i1_v], rb_v, sem).wait()
            for r in range(_CH):
                g0b = g0_v[r, :]
                g1b = g1_v[r, :]

                def body(j, carry, r=r, g0b=g0b, g1b=g1b):
                    sl = pl.ds(pl.multiple_of(j * 16, 16), 16)
                    o_v[r, sl] = ra_v[r, sl] * g0b + rb_v[r, sl] * g1b
                    return carry

                lax.fori_loop(0, DIM // 16, body, 0)
            pltpu.sync_copy(o_v, out_hbm.at[pl.ds(tb, _CH)])

    return k(ys, dst, g16)


@jax.jit
def kernel(x, r_g, r_b, r_w, r_wb, ln_g, ln_b, W1, b1, W2, b2):
    shp = x.shape
    xf = x.reshape(T, DIM)
    xhat, idx2, wn = _router_call(xf, r_g, r_b, r_w, r_wb)
    dst32, meta = _plan_call(idx2.reshape(_PR, _PC))
    sarr = jnp.concatenate([meta[0, :NT], meta[1, :NT]])
    dst = dst32.reshape(NP)
    xs = _sc_scatter(xhat, dst)
    ys = _grouped_call(sarr, xs, ln_g, ln_b,
                       W1.astype(jnp.bfloat16), b1,
                       W2.astype(jnp.bfloat16), b2)
    g16 = jnp.broadcast_to(wn.reshape(NP)[:, None], (NP, 16))
    out = _sc_combine(ys, dst, g16)
    return out.reshape(shp)


# speedup vs baseline: 1.2766x; 1.2766x over previous
import functools
import math

import jax
import jax.numpy as jnp
from jax import lax
from jax.experimental import pallas as pl
from jax.experimental.pallas import tpu as pltpu
from jax.experimental.pallas import tpu_sc as plsc

DIM = 1024
E = 8
K = 2
HID = 4096
T = 2048
TT = 128
NTT = T // TT
TTG = 256
HB = 1024
NHB = HID // HB
NP = K * T
NT = 24
NR = NT * TTG
EPS = 1e-6
_INV_SQRT2 = 1.0 / math.sqrt(2.0)

_NC = 2
_NW = 32
_TOKW = T // _NW
_CH = 16


def _router_body(x_ref, rg_ref, rb_ref, rw_ref, rwb_ref,
                 xhat_ref, idx_ref, wn_ref):
    x = x_ref[...]
    mu = jnp.mean(x, axis=1, keepdims=True)
    xc = x - mu
    var = jnp.mean(xc * xc, axis=1, keepdims=True)
    xh = xc * lax.rsqrt(var + EPS)
    xhat_ref[...] = xh
    xr = xh * rg_ref[...] + rb_ref[...]
    l8 = jax.lax.dot_general(rw_ref[...], xr, (((1,), (1,)), ((), ())),
                             preferred_element_type=jnp.float32)
    l8 = l8 + rwb_ref[...]
    m = jnp.max(l8, axis=0, keepdims=True)
    pe = jnp.exp(l8 - m)
    p = pe / jnp.sum(pe, axis=0, keepdims=True)
    io = lax.broadcasted_iota(jnp.int32, (E, TT), 0)
    m1 = jnp.max(p, axis=0, keepdims=True)
    i1 = jnp.min(jnp.where(p >= m1, io, E), axis=0, keepdims=True)
    p2 = jnp.where(io == i1, -jnp.inf, p)
    m2 = jnp.max(p2, axis=0, keepdims=True)
    i2 = jnp.min(jnp.where(p2 >= m2, io, E), axis=0, keepdims=True)
    s = m1 + m2
    idx_ref[...] = jnp.concatenate([i1, i2], axis=0)
    wn_ref[...] = jnp.concatenate([m1 / s, m2 / s], axis=0)


def _router_call(xf, r_g, r_b, r_w, r_wb):
    return pl.pallas_call(
        _router_body,
        grid=(NTT,),
        in_specs=[
            pl.BlockSpec((TT, DIM), lambda t: (t, 0)),
            pl.BlockSpec((1, DIM), lambda t: (0, 0)),
            pl.BlockSpec((1, DIM), lambda t: (0, 0)),
            pl.BlockSpec((E, DIM), lambda t: (0, 0)),
            pl.BlockSpec((E, 1), lambda t: (0, 0)),
        ],
        out_specs=[
            pl.BlockSpec((TT, DIM), lambda t: (t, 0)),
            pl.BlockSpec((K, TT), lambda t: (0, t)),
            pl.BlockSpec((K, TT), lambda t: (0, t)),
        ],
        out_shape=[
            jax.ShapeDtypeStruct((T, DIM), jnp.float32),
            jax.ShapeDtypeStruct((K, T), jnp.int32),
            jax.ShapeDtypeStruct((K, T), jnp.float32),
        ],
        compiler_params=pltpu.CompilerParams(
            dimension_semantics=("arbitrary",)),
    )(xf, r_g.reshape(1, DIM), r_b.reshape(1, DIM), r_w, r_wb.reshape(E, 1))


_PR = 32
_PC = NP // _PR


def _plan_body(ep_ref, dst_ref, meta_ref):
    ep = ep_ref[...]
    hp = jax.lax.Precision.HIGHEST
    ltri = (lax.broadcasted_iota(jnp.int32, (_PC, _PC), 0)
            <= lax.broadcasted_iota(jnp.int32, (_PC, _PC), 1)
            ).astype(jnp.float32)
    strt = (lax.broadcasted_iota(jnp.int32, (_PR, _PR), 1)
            < lax.broadcasted_iota(jnp.int32, (_PR, _PR), 0)
            ).astype(jnp.float32)
    dst = jnp.zeros((_PR, _PC), jnp.float32)
    cnts = []
    for e in range(E):
        m_e = (ep == e).astype(jnp.float32)
        local = jax.lax.dot_general(m_e, ltri, (((1,), (0,)), ((), ())),
                                    precision=hp,
                                    preferred_element_type=jnp.float32)
        rowsum = jnp.sum(m_e, axis=1, keepdims=True)
        carry = jax.lax.dot_general(strt, rowsum, (((1,), (0,)), ((), ())),
                                    precision=hp,
                                    preferred_element_type=jnp.float32)
        cnts.append(jnp.sum(rowsum))
        dst = dst + m_e * (local - 1.0 + carry)
    off = []
    run = jnp.float32(0.0)
    total = jnp.float32(0.0)
    elast = jnp.float32(0.0)
    for e in range(E):
        off.append(run)
        pc = jnp.ceil(cnts[e] * (1.0 / TTG)) * TTG
        run = run + pc
        total = total + pc
        elast = jnp.where(cnts[e] > 0, jnp.float32(e), elast)
    for e in range(E):
        m_e = (ep == e).astype(jnp.float32)
        dst = dst + m_e * off[e]
    dst_ref[...] = dst.astype(jnp.int32)
    starts = lax.broadcasted_iota(jnp.int32, (1, TT), 1) * TTG
    te = jnp.zeros((1, TT), jnp.int32)
    for e in range(E):
        te = te + (starts >= off[e].astype(jnp.int32)).astype(jnp.int32)
    te = jnp.minimum(te - 1, elast.astype(jnp.int32))
    tv = (starts < total.astype(jnp.int32)).astype(jnp.int32)
    meta_ref[...] = jnp.concatenate([te, tv], axis=0)


def _plan_call(e_pairs):
    return pl.pallas_call(
        _plan_body,
        grid=(1,),
        in_specs=[pl.BlockSpec((_PR, _PC), lambda i: (0, 0))],
        out_specs=[
            pl.BlockSpec((_PR, _PC), lambda i: (0, 0)),
            pl.BlockSpec((2, TT), lambda i: (0, 0)),
        ],
        out_shape=[
            jax.ShapeDtypeStruct((_PR, _PC), jnp.int32),
            jax.ShapeDtypeStruct((2, TT), jnp.int32),
        ],
        compiler_params=pltpu.CompilerParams(
            dimension_semantics=("arbitrary",)),
    )(e_pairs)


def _sc_scatter(xhat, dst):
    mesh = plsc.VectorSubcoreMesh(core_axis_name="c", subcore_axis_name="s")

    @functools.partial(
        pl.kernel, mesh=mesh,
        out_type=jax.ShapeDtypeStruct((NR, DIM), jnp.float32),
        scratch_types=[
            pltpu.VMEM((_CH, DIM), jnp.float32),
            pltpu.VMEM((_CH,), jnp.int32),
            pltpu.VMEM((_CH,), jnp.int32),
            pltpu.SemaphoreType.DMA,
        ],
    )
    def k(xhat_hbm, dst_hbm, xs_hbm, rows_v, i0_v, i1_v, sem):
        wid = lax.axis_index("s") * _NC + lax.axis_index("c")
        base = wid * _TOKW
        for c in range(_TOKW // _CH):
            tb = base + c * _CH
            pltpu.sync_copy(xhat_hbm.at[pl.ds(tb, _CH)], rows_v)
            pltpu.sync_copy(dst_hbm.at[pl.ds(tb, _CH)], i0_v)
            pltpu.sync_copy(dst_hbm.at[pl.ds(T + tb, _CH)], i1_v)
            pltpu.async_copy(rows_v, xs_hbm.at[i0_v], sem).wait()
            pltpu.async_copy(rows_v, xs_hbm.at[i1_v], sem).wait()

    return k(xhat, dst)


def _grouped_body(s_ref, xs_ref, w1_ref, b1_ref, w2_ref, b2_ref,
                  lng_ref, lnb_ref, ys_ref):
    h = pl.program_id(0)
    t = pl.program_id(1)
    rows = pl.ds(t * TTG, TTG)

    @pl.when(s_ref[NT + t] != 0)
    def _compute():
        xt = xs_ref[...]
        xln = (xt * lng_ref[0] + lnb_ref[0]).astype(jnp.bfloat16)
        hm = jax.lax.dot_general(xln, w1_ref[0].astype(jnp.bfloat16),
                                 (((1,), (1,)), ((), ())),
                                 preferred_element_type=jnp.float32)
        hm = hm + b1_ref[0]
        hg = 0.5 * hm * (1.0 + lax.erf(hm * _INV_SQRT2))
        contrib = jax.lax.dot_general(hg.astype(jnp.bfloat16),
                                      w2_ref[0].astype(jnp.bfloat16),
                                      (((1,), (1,)), ((), ())),
                                      preferred_element_type=jnp.float32)

        @pl.when(h == 0)
        def _init():
            ys_ref[rows, :] = contrib + b2_ref[0]

        @pl.when(h != 0)
        def _acc():
            ys_ref[rows, :] = ys_ref[rows, :] + contrib


def _grouped_call(sarr, xs, ln_g, ln_b, W1, b1, W2, b2):
    grid_spec = pltpu.PrefetchScalarGridSpec(
        num_scalar_prefetch=1,
        grid=(NHB, NT),
        in_specs=[
            pl.BlockSpec((TTG, DIM), lambda h, t, s: (t, 0)),
            pl.BlockSpec((1, HB, DIM), lambda h, t, s: (s[t], h, 0)),
            pl.BlockSpec((1, 1, HB), lambda h, t, s: (s[t], 0, h)),
            pl.BlockSpec((1, DIM, HB), lambda h, t, s: (s[t], 0, h)),
            pl.BlockSpec((1, 1, DIM), lambda h, t, s: (s[t], 0, 0)),
            pl.BlockSpec((1, 1, DIM), lambda h, t, s: (s[t], 0, 0)),
            pl.BlockSpec((1, 1, DIM), lambda h, t, s: (s[t], 0, 0)),
        ],
        out_specs=pl.BlockSpec((NR, DIM), lambda h, t, s: (0, 0)),
    )
    return pl.pallas_call(
        _grouped_body,
        grid_spec=grid_spec,
        out_shape=jax.ShapeDtypeStruct((NR, DIM), jnp.float32),
        compiler_params=pltpu.CompilerParams(
            dimension_semantics=("arbitrary", "arbitrary")),
    )(sarr, xs, W1, b1.reshape(E, 1, HID), W2, b2.reshape(E, 1, DIM),
      ln_g.reshape(E, 1, DIM), ln_b.reshape(E, 1, DIM))


def _sc_combine(ys, dst, g16):
    mesh = plsc.VectorSubcoreMesh(core_axis_name="c", subcore_axis_name="s")

    @functools.partial(
        pl.kernel, mesh=mesh,
        out_type=jax.ShapeDtypeStruct((T, DIM), jnp.float32),
        scratch_types=[
            pltpu.VMEM((_CH, DIM), jnp.float32),
            pltpu.VMEM((_CH, DIM), jnp.float32),
            pltpu.VMEM((_CH, DIM), jnp.float32),
            pltpu.VMEM((_CH,), jnp.int32),
            pltpu.VMEM((_CH,), jnp.int32),
            pltpu.VMEM((_CH, 16), jnp.float32),
            pltpu.VMEM((_CH, 16), jnp.float32),
            pltpu.SemaphoreType.DMA,
        ],
    )
    def k(ys_hbm, dst_hbm, g_hbm, out_hbm,
          ra_v, rb_v, o_v, i0_v, i1_v, g0_v, g1_v, sem):
        wid = lax.axis_index("s") * _NC + lax.axis_index("c")
        base = wid * _TOKW
        for c in range(_TOKW // _CH):
            tb = base + c * _CH
            pltpu.sync_copy(dst_hbm.at[pl.ds(tb, _CH)], i0_v)
            pltpu.sync_copy(dst_hbm.at[pl.ds(T + tb, _CH)], i1_v)
            pltpu.sync_copy(g_hbm.at[pl.ds(tb, _CH)], g0_v)
            pltpu.sync_copy(g_hbm.at[pl.ds(T + tb, _CH)], g1_v)
            ha = pltpu.async_copy(ys_hbm.at[i0_v], ra_v, sem)
            hb = pltpu.async_copy(ys_hbm.at[i1_v], rb_v, sem)
            ha.wait()
            hb.wait()
            for r in range(_CH):
                g0b = g0_v[r, :]
                g1b = g1_v[r, :]

                def body(j, carry, r=r, g0b=g0b, g1b=g1b):
                    for u in range(8):
                        sl = pl.ds(pl.multiple_of(j * 128, 128) + u * 16, 16)
                        o_v[r, sl] = ra_v[r, sl] * g0b + rb_v[r, sl] * g1b
                    return carry

                lax.fori_loop(0, DIM // 128, body, 0)
            pltpu.sync_copy(o_v, out_hbm.at[pl.ds(tb, _CH)])

    return k(ys, dst, g16)


@jax.jit
def kernel(x, r_g, r_b, r_w, r_wb, ln_g, ln_b, W1, b1, W2, b2):
    shp = x.shape
    xf = x.reshape(T, DIM)
    xhat, idx2, wn = _router_call(xf, r_g, r_b, r_w, r_wb)
    dst32, meta = _plan_call(idx2.reshape(_PR, _PC))
    sarr = jnp.concatenate([meta[0, :NT], meta[1, :NT]])
    dst = dst32.reshape(NP)
    xs = _sc_scatter(xhat, dst)
    ys = _grouped_call(sarr, xs, ln_g, ln_b, W1, b1, W2, b2)
    g16 = jnp.broadcast_to(wn.reshape(NP)[:, None], (NP, 16))
    out = _sc_combine(ys, dst, g16)
    return out.reshape(shp)
